# Initial kernel scaffold; baseline (speedup 1.0000x reference)
import jax
import jax.numpy as jnp
from jax.experimental import pallas as pl


def kernel(x, edge_index, W1, b1, W2, att_src, att_dst, b2, W3, b3):
    raise NotImplementedError("write your pallas kernel here")


# speedup vs baseline: 19.2020x; 19.2020x over previous
import functools

import jax
import jax.numpy as jnp
from jax import lax
from jax.experimental import pallas as pl
from jax.experimental.pallas import tpu as pltpu
from jax.experimental.pallas import tpu_sc as plsc

N = 10000
E = 320000
NC = 2
NS = 16
CHUNK = 80
ROW_BLK = 1000
F32 = jnp.float32
PREC = lax.Precision.HIGHEST


def _mesh():
    return plsc.VectorSubcoreMesh(core_axis_name="c", subcore_axis_name="s")


_SC_PARAMS = pltpu.CompilerParams(use_tc_tiling_on_sc=False,
                                  needs_layout_passes=False)


def _mm_kernel(x_ref, w_ref, o_ref):
    o_ref[...] = jnp.dot(x_ref[...], w_ref[...], precision=PREC,
                         preferred_element_type=F32)


def _matmul(x, w):
    n, k = x.shape
    m = w.shape[1]
    return pl.pallas_call(
        _mm_kernel,
        grid=(n // ROW_BLK,),
        in_specs=[pl.BlockSpec((ROW_BLK, k), lambda i: (i, 0)),
                  pl.BlockSpec((k, m), lambda i: (0, 0))],
        out_specs=pl.BlockSpec((ROW_BLK, m), lambda i: (i, 0)),
        out_shape=jax.ShapeDtypeStruct((n, m), F32),
    )(x, w)


def _prescale_kernel(degp_ref, xw_ref, xws_ref, dinv_ref):
    deg = 1.0 + degp_ref[0, :, 0:1] + degp_ref[1, :, 0:1]
    dinv = lax.rsqrt(deg)
    dinv_ref[...] = dinv
    xws_ref[...] = dinv * xw_ref[...]


def _prescale(degp, xw):
    n, m = xw.shape
    return pl.pallas_call(
        _prescale_kernel,
        grid=(n // ROW_BLK,),
        in_specs=[pl.BlockSpec((NC, ROW_BLK, 16), lambda i: (0, i, 0)),
                  pl.BlockSpec((ROW_BLK, m), lambda i: (i, 0))],
        out_specs=[pl.BlockSpec((ROW_BLK, m), lambda i: (i, 0)),
                   pl.BlockSpec((ROW_BLK, 1), lambda i: (i, 0))],
        out_shape=[jax.ShapeDtypeStruct((n, m), F32),
                   jax.ShapeDtypeStruct((n, 1), F32)],
    )(degp, xw)


def _gcn_post_kernel(acc_ref, dinv_ref, b_ref, o_ref, *, act):
    v = dinv_ref[...] * (acc_ref[0] + acc_ref[1]) + b_ref[...][None, :]
    if act:
        v = jnp.maximum(v, 0.0)
    o_ref[...] = v


def _gcn_post(acc, dinv, b, act):
    n, m = acc.shape[1], acc.shape[2]
    return pl.pallas_call(
        functools.partial(_gcn_post_kernel, act=act),
        grid=(n // ROW_BLK,),
        in_specs=[pl.BlockSpec((NC, ROW_BLK, m), lambda i: (0, i, 0)),
                  pl.BlockSpec((ROW_BLK, 1), lambda i: (i, 0)),
                  pl.BlockSpec((m,), lambda i: (0,))],
        out_specs=pl.BlockSpec((ROW_BLK, m), lambda i: (i, 0)),
        out_shape=jax.ShapeDtypeStruct((n, m), F32),
    )(acc, dinv, b)


def _gat_mm_kernel(h_ref, w_ref, aa_ref, xw_ref, a_ref):
    xw = jnp.dot(h_ref[...], w_ref[...], precision=PREC,
                 preferred_element_type=F32)
    xw_ref[...] = xw
    a_ref[...] = jnp.dot(xw, aa_ref[...], precision=PREC,
                         preferred_element_type=F32)


def _gat_mm(h, w, aa):
    n, k = h.shape
    m = w.shape[1]
    return pl.pallas_call(
        _gat_mm_kernel,
        grid=(n // ROW_BLK,),
        in_specs=[pl.BlockSpec((ROW_BLK, k), lambda i: (i, 0)),
                  pl.BlockSpec((k, m), lambda i: (0, 0)),
                  pl.BlockSpec((m, 8), lambda i: (0, 0))],
        out_specs=[pl.BlockSpec((ROW_BLK, m), lambda i: (i, 0)),
                   pl.BlockSpec((ROW_BLK, 8), lambda i: (i, 0))],
        out_shape=[jax.ShapeDtypeStruct((n, m), F32),
                   jax.ShapeDtypeStruct((n, 8), F32)],
    )(h, w, aa)


def _pack_kernel(xw_ref, a_ref, xwz_ref, zd_ref):
    zpad = jnp.zeros((ROW_BLK, 14), F32)
    for h in range(2):
        xwz_ref[h] = jnp.concatenate(
            [xw_ref[:, h * 128:(h + 1) * 128], a_ref[:, 2 * h:2 * h + 2], zpad],
            axis=1)
        zd_ref[h] = jnp.concatenate([a_ref[:, 4 + 2 * h:6 + 2 * h], zpad],
                                    axis=1)


def _pack(xw, a):
    n = xw.shape[0]
    return pl.pallas_call(
        _pack_kernel,
        grid=(n // ROW_BLK,),
        in_specs=[pl.BlockSpec((ROW_BLK, 256), lambda i: (i, 0)),
                  pl.BlockSpec((ROW_BLK, 8), lambda i: (i, 0))],
        out_specs=[pl.BlockSpec((NC, ROW_BLK, 144), lambda i: (0, i, 0)),
                   pl.BlockSpec((NC, ROW_BLK, 16), lambda i: (0, i, 0))],
        out_shape=[jax.ShapeDtypeStruct((NC, n, 144), F32),
                   jax.ShapeDtypeStruct((NC, n, 16), F32)],
    )(xw, a)


def _gat_post_kernel(accf_ref, den_ref, xw_ref, a_ref, b_ref, o_ref):
    a = a_ref[...]
    z = a[:, 0:4] + a[:, 4:8]
    exs = jnp.exp(jnp.maximum(z, 0.2 * z))
    for h in range(4):
        eh = exs[:, h:h + 1]
        num = accf_ref[:, h * 64:(h + 1) * 64] + eh * xw_ref[:, h * 64:(h + 1) * 64]
        dh = jnp.maximum(den_ref[:, h:h + 1] + eh, 1e-16)
        v = num / dh + b_ref[...][None, h * 64:(h + 1) * 64]
        o_ref[:, h * 64:(h + 1) * 64] = jnp.where(v > 0, v, jnp.exp(jnp.minimum(v, 0.0)) - 1.0)


def _gat_post(accf, den, xw, a, b):
    n = accf.shape[0]
    return pl.pallas_call(
        _gat_post_kernel,
        grid=(n // ROW_BLK,),
        in_specs=[pl.BlockSpec((ROW_BLK, 256), lambda i: (i, 0)),
                  pl.BlockSpec((ROW_BLK, 4), lambda i: (i, 0)),
                  pl.BlockSpec((ROW_BLK, 256), lambda i: (i, 0)),
                  pl.BlockSpec((ROW_BLK, 8), lambda i: (i, 0)),
                  pl.BlockSpec((256,), lambda i: (0,))],
        out_specs=pl.BlockSpec((ROW_BLK, 256), lambda i: (i, 0)),
        out_shape=jax.ShapeDtypeStruct((n, 256), F32),
    )(accf, den, xw, a, b)


def _mm_scale_kernel(x_ref, w_ref, dinv_ref, o_ref):
    o_ref[...] = dinv_ref[...] * jnp.dot(x_ref[...], w_ref[...], precision=PREC,
                                         preferred_element_type=F32)


def _mm_scale(x, w, dinv):
    n, k = x.shape
    m = w.shape[1]
    return pl.pallas_call(
        _mm_scale_kernel,
        grid=(n // ROW_BLK,),
        in_specs=[pl.BlockSpec((ROW_BLK, k), lambda i: (i, 0)),
                  pl.BlockSpec((k, m), lambda i: (0, 0)),
                  pl.BlockSpec((ROW_BLK, 1), lambda i: (i, 0))],
        out_specs=pl.BlockSpec((ROW_BLK, m), lambda i: (i, 0)),
        out_shape=jax.ShapeDtypeStruct((n, m), F32),
    )(x, w, dinv)


def _sc_deg(dst, zeros16):
    epw = E // (NC * NS)
    nchunk = epw // CHUNK

    @functools.partial(
        pl.kernel, mesh=_mesh(), compiler_params=_SC_PARAMS,
        out_type=jax.ShapeDtypeStruct((NC, N, 16), F32),
        scratch_types=[
            pltpu.VMEM((CHUNK,), jnp.int32),
            pltpu.VMEM((CHUNK, 16), F32),
            pltpu.VMEM_SHARED((N, 16), F32),
            pltpu.SemaphoreType.DMA,
        ])
    def k(dst_hbm, z_hbm, out_hbm, idx_d, ones_v, acc_sh, sem):
        c = lax.axis_index("c")
        s = lax.axis_index("s")
        one = jnp.full((16,), 1.0, F32)

        @pl.loop(0, CHUNK)
        def _(r):
            ones_v[r] = one

        @pl.when(s == 0)
        def _():
            pltpu.sync_copy(z_hbm, acc_sh)

        plsc.subcore_barrier()
        base0 = (c * NS + s) * epw

        @pl.loop(0, nchunk)
        def _(j):
            base = base0 + j * CHUNK
            pltpu.sync_copy(dst_hbm.at[pl.ds(base, CHUNK)], idx_d)
            pltpu.sync_copy(ones_v, acc_sh.at[idx_d], add=True)

        plsc.subcore_barrier()

        @pl.when(s == 0)
        def _():
            pltpu.sync_copy(acc_sh, out_hbm.at[c])

    return k(dst, zeros16)


def _sc_gcn(xws, src, dst, zeros64):
    epw = E // (NC * NS)
    nchunk = epw // CHUNK

    @functools.partial(
        pl.kernel, mesh=_mesh(), compiler_params=_SC_PARAMS,
        out_type=jax.ShapeDtypeStruct((NC, N, 64), F32),
        scratch_types=[
            pltpu.VMEM((CHUNK,), jnp.int32),
            pltpu.VMEM((CHUNK,), jnp.int32),
            pltpu.VMEM((CHUNK, 64), F32),
            pltpu.VMEM_SHARED((N, 64), F32),
            pltpu.SemaphoreType.DMA,
        ])
    def k(xws_hbm, src_hbm, dst_hbm, z_hbm, out_hbm,
          idx_s, idx_d, rows, acc_sh, sem):
        c = lax.axis_index("c")
        s = lax.axis_index("s")

        @pl.when(jnp.logical_and(s == 0, c == 0))
        def _():
            pltpu.sync_copy(xws_hbm, acc_sh)

        @pl.when(jnp.logical_and(s == 0, c == 1))
        def _():
            pltpu.sync_copy(z_hbm, acc_sh)

        plsc.subcore_barrier()
        base0 = (c * NS + s) * epw

        @pl.loop(0, nchunk)
        def _(j):
            base = base0 + j * CHUNK
            pltpu.sync_copy(src_hbm.at[pl.ds(base, CHUNK)], idx_s)
            pltpu.sync_copy(dst_hbm.at[pl.ds(base, CHUNK)], idx_d)
            pltpu.async_copy(xws_hbm.at[idx_s], rows, sem).wait()
            pltpu.sync_copy(rows, acc_sh.at[idx_d], add=True)

        plsc.subcore_barrier()

        @pl.when(s == 0)
        def _():
            pltpu.sync_copy(acc_sh, out_hbm.at[c])

    return k(xws, src, dst, zeros64)


def _sc_gat(xwz, zd, src, dst, zeros144):
    epw = E // NS
    nchunk = epw // CHUNK

    @functools.partial(
        pl.kernel, mesh=_mesh(), compiler_params=_SC_PARAMS,
        out_type=jax.ShapeDtypeStruct((NC, N, 144), F32),
        scratch_types=[
            pltpu.VMEM((CHUNK,), jnp.int32),
            pltpu.VMEM((CHUNK,), jnp.int32),
            pltpu.VMEM((CHUNK,), jnp.int32),
            pltpu.VMEM((CHUNK,), jnp.int32),
            pltpu.VMEM((CHUNK, 144), F32),
            pltpu.VMEM((CHUNK, 144), F32),
            pltpu.VMEM((CHUNK, 16), F32),
            pltpu.VMEM_SHARED((N, 144), F32),
            pltpu.SemaphoreType.DMA,
        ])
    def k(xwz_hbm, zd_hbm, src_hbm, dst_hbm, z_hbm, out_hbm,
          idx_s, idx_d, idx_so, idx_do, frows, srows, zdst, acc_sh, sem):
        c = lax.axis_index("c")
        s = lax.axis_index("s")

        @pl.when(s == 0)
        def _():
            pltpu.sync_copy(z_hbm, acc_sh)

        plsc.subcore_barrier()
        offv = jnp.full((16,), c * N, jnp.int32)
        lanes = lax.iota(jnp.int32, 16)
        mask01 = lanes < 2
        zero16 = jnp.zeros((16,), F32)
        base0 = s * epw

        @pl.loop(0, nchunk)
        def _(j):
            base = base0 + j * CHUNK
            pltpu.sync_copy(src_hbm.at[pl.ds(base, CHUNK)], idx_s)
            pltpu.sync_copy(dst_hbm.at[pl.ds(base, CHUNK)], idx_d)

            @pl.loop(0, CHUNK // 16)
            def _(t):
                sl = pl.ds(t * 16, 16)
                idx_so[sl] = idx_s[sl] + offv
                idx_do[sl] = idx_d[sl] + offv

            pltpu.async_copy(xwz_hbm.at[idx_so], frows, sem).wait()
            pltpu.async_copy(zd_hbm.at[idx_do], zdst, sem).wait()

            @pl.loop(0, CHUNK)
            def _(r):
                z = frows[r, pl.ds(128, 16)] + zdst[r]
                ex = jnp.exp(jnp.maximum(z, 0.2 * z))
                srows[r, pl.ds(128, 16)] = jnp.where(mask01, ex, zero16)
                rsplat = jnp.full((16,), r, jnp.int32)
                ex0 = plsc.load_gather(srows, [rsplat,
                                               jnp.full((16,), 128, jnp.int32)])
                ex1 = plsc.load_gather(srows, [rsplat,
                                               jnp.full((16,), 129, jnp.int32)])
                for t in range(4):
                    sl = pl.ds(t * 16, 16)
                    srows[r, sl] = frows[r, sl] * ex0
                for t in range(4, 8):
                    sl = pl.ds(t * 16, 16)
                    srows[r, sl] = frows[r, sl] * ex1

            pltpu.sync_copy(srows, acc_sh.at[idx_d], add=True)

        plsc.subcore_barrier()

        @pl.when(s == 0)
        def _():
            pltpu.sync_copy(acc_sh, out_hbm.at[c])

    return k(xwz, zd, src, dst, zeros144)


def kernel(x, edge_index, W1, b1, W2, att_src, att_dst, b2, W3, b3):
    src = edge_index[0]
    dst = edge_index[1]

    eye = jnp.eye(4, dtype=F32)
    asrc_m = (att_src[:, :, None] * eye[:, None, :]).reshape(256, 4)
    adst_m = (att_dst[:, :, None] * eye[:, None, :]).reshape(256, 4)
    aa = jnp.concatenate([asrc_m, adst_m], axis=1)

    z16 = jnp.zeros((N, 16), F32)
    z64 = jnp.zeros((N, 64), F32)
    z144 = jnp.zeros((N, 144), F32)

    degp = _sc_deg(dst, z16)
    xw1 = _matmul(x, W1)
    xws1, dinv = _prescale(degp, xw1)

    acc1 = _sc_gcn(xws1, src, dst, z64)
    h1 = _gcn_post(acc1, dinv, b1, act=True)

    xw2, a = _gat_mm(h1, W2, aa)
    xwz, zd = _pack(xw2, a)
    acc2 = _sc_gat(xwz.reshape(NC * N, 144), zd.reshape(NC * N, 16),
                   src, dst, z144)
    accf = jnp.concatenate([acc2[0, :, :128], acc2[1, :, :128]], axis=1)
    den = jnp.concatenate([acc2[0, :, 128:130], acc2[1, :, 128:130]], axis=1)
    h2 = _gat_post(accf, den, xw2, a, b2)

    xws3 = _mm_scale(h2, W3, dinv)
    acc3 = _sc_gcn(xws3, src, dst, z64)
    return _gcn_post(acc3, dinv, b3, act=False)
